# Initial kernel scaffold; baseline (speedup 1.0000x reference)
import jax
import jax.numpy as jnp
from jax.experimental import pallas as pl


def kernel(inputs, table):
    raise NotImplementedError("write your pallas kernel here")


# speedup vs baseline: 13.6527x; 13.6527x over previous
import functools

import jax
import jax.numpy as jnp
from jax.experimental import pallas as pl
from jax.experimental.pallas import tpu as pltpu

MAXV = 147
KPAD = 160


def _body(idx_ref, tab_ref, out_ref, *, t_imgs, n_chan, kpad, pixels):
    iota = jax.lax.broadcasted_iota(jnp.int32, (kpad, pixels), 0)
    iota_bf = iota.astype(jnp.bfloat16)
    for t in range(t_imgs):
        acc = None
        for c in range(n_chan):
            idx_bf = idx_ref[t, c, :].astype(jnp.bfloat16)
            onehot = (iota_bf == idx_bf[None, :]).astype(jnp.bfloat16)
            part = jnp.dot(tab_ref[c], onehot,
                           preferred_element_type=jnp.float32)
            acc = part if acc is None else acc + part
        out_ref[t] = acc


@jax.jit
def kernel(inputs, table):
    B, C, H, W = inputs.shape
    V, D = table.shape
    P = H * W
    maxv = V // C

    idx = inputs.astype(jnp.int32).reshape(B, C, P)

    tab = table.reshape(C, maxv, D)
    tab = jnp.pad(tab, ((0, 0), (0, KPAD - maxv), (0, 0)))
    tabT = jnp.transpose(tab, (0, 2, 1)).astype(jnp.bfloat16)

    T = 8
    grid = (B // T,)
    out = pl.pallas_call(
        functools.partial(_body, t_imgs=T, n_chan=C, kpad=KPAD, pixels=P),
        grid=grid,
        in_specs=[
            pl.BlockSpec((T, C, P), lambda i: (i, 0, 0)),
            pl.BlockSpec((C, D, KPAD), lambda i: (0, 0, 0)),
        ],
        out_specs=pl.BlockSpec((T, D, P), lambda i: (i, 0, 0)),
        out_shape=jax.ShapeDtypeStruct((B, D, P), jnp.float32),
    )(idx, tabT)
    return out.reshape(B, D, H, W)


# speedup vs baseline: 13.7588x; 1.0078x over previous
import functools

import jax
import jax.numpy as jnp
from jax.experimental import pallas as pl
from jax.experimental.pallas import tpu as pltpu

MAXV = 147
KPAD = 160


def _body(idx_ref, tab_ref, out_ref, *, t_imgs, n_chan, kpad, pixels):
    iota = jax.lax.broadcasted_iota(jnp.int32, (kpad, pixels), 0)
    iota_bf = iota.astype(jnp.bfloat16)
    for t in range(t_imgs):
        hots = []
        for c in range(n_chan):
            idx_bf = idx_ref[t, c, :].astype(jnp.bfloat16)
            hots.append((iota_bf == idx_bf[None, :]).astype(jnp.bfloat16))
        onehot = jnp.concatenate(hots, axis=0)
        out_ref[t] = jnp.dot(tab_ref[...], onehot,
                             preferred_element_type=jnp.float32)


@jax.jit
def kernel(inputs, table):
    B, C, H, W = inputs.shape
    V, D = table.shape
    P = H * W
    maxv = V // C

    idx = inputs.astype(jnp.int32).reshape(B, C, P)

    tab = table.reshape(C, maxv, D)
    tab = jnp.pad(tab, ((0, 0), (0, KPAD - maxv), (0, 0)))
    tabT = jnp.transpose(tab, (2, 0, 1)).reshape(D, C * KPAD)
    tabT = tabT.astype(jnp.bfloat16)

    T = 8
    grid = (B // T,)
    out = pl.pallas_call(
        functools.partial(_body, t_imgs=T, n_chan=C, kpad=KPAD, pixels=P),
        grid=grid,
        in_specs=[
            pl.BlockSpec((T, C, P), lambda i: (i, 0, 0)),
            pl.BlockSpec((D, C * KPAD), lambda i: (0, 0)),
        ],
        out_specs=pl.BlockSpec((T, D, P), lambda i: (i, 0, 0)),
        out_shape=jax.ShapeDtypeStruct((B, D, P), jnp.float32),
        compiler_params=pltpu.CompilerParams(
            dimension_semantics=("parallel",)),
    )(idx, tabT)
    return out.reshape(B, D, H, W)


# speedup vs baseline: 17.8990x; 1.3009x over previous
import functools

import jax
import jax.numpy as jnp
from jax.experimental import pallas as pl
from jax.experimental.pallas import tpu as pltpu

MAXV = 147
KPAD = 160


def _body(idx_ref, tab_ref, out_ref, *, t_imgs, n_chan, kpad, pixels):
    iota = jax.lax.broadcasted_iota(jnp.int32, (kpad, pixels), 0)
    iota_bf = iota.astype(jnp.bfloat16)
    one = jnp.bfloat16(1.0)
    zero = jnp.bfloat16(0.0)
    for t in range(t_imgs):
        hots = []
        for c in range(n_chan):
            idx_bf = idx_ref[t, c, :].astype(jnp.bfloat16)
            d = iota_bf - idx_bf[None, :]
            hots.append(jnp.maximum(one - jnp.abs(d), zero))
        onehot = jnp.concatenate(hots, axis=0)
        out_ref[t] = jnp.dot(tab_ref[...], onehot,
                             preferred_element_type=jnp.float32)


@jax.jit
def kernel(inputs, table):
    B, C, H, W = inputs.shape
    V, D = table.shape
    P = H * W
    maxv = V // C

    idx = inputs.astype(jnp.int32).reshape(B, C, P)

    tab = table.reshape(C, maxv, D)
    tab = jnp.pad(tab, ((0, 0), (0, KPAD - maxv), (0, 0)))
    tabT = jnp.transpose(tab, (2, 0, 1)).reshape(D, C * KPAD)
    tabT = tabT.astype(jnp.bfloat16)

    T = 8
    grid = (B // T,)
    out = pl.pallas_call(
        functools.partial(_body, t_imgs=T, n_chan=C, kpad=KPAD, pixels=P),
        grid=grid,
        in_specs=[
            pl.BlockSpec((T, C, P), lambda i: (i, 0, 0)),
            pl.BlockSpec((D, C * KPAD), lambda i: (0, 0)),
        ],
        out_specs=pl.BlockSpec((T, D, P), lambda i: (i, 0, 0)),
        out_shape=jax.ShapeDtypeStruct((B, D, P), jnp.float32),
        compiler_params=pltpu.CompilerParams(
            dimension_semantics=("parallel",)),
    )(idx, tabT)
    return out.reshape(B, D, H, W)


# speedup vs baseline: 18.6398x; 1.0414x over previous
import functools

import jax
import jax.numpy as jnp
from jax.experimental import pallas as pl
from jax.experimental.pallas import tpu as pltpu

MAXV = 147
KPAD = 160


def _body(idx_ref, tab_ref, out_ref, *, t_imgs, n_chan, kpad, pixels):
    iota = jax.lax.broadcasted_iota(jnp.int32, (kpad, pixels), 0)
    iota_bf = iota.astype(jnp.bfloat16)
    one = jnp.bfloat16(1.0)
    zero = jnp.bfloat16(0.0)
    for t in range(t_imgs):
        hots = []
        for c in range(n_chan):
            idx_bf = idx_ref[t, c, :].astype(jnp.bfloat16)
            d = iota_bf - idx_bf[None, :]
            hots.append(jnp.maximum(one - jnp.abs(d), zero))
        onehot = jnp.concatenate(hots, axis=0)
        out_ref[t] = jnp.dot(tab_ref[...], onehot,
                             preferred_element_type=jnp.float32)


@jax.jit
def kernel(inputs, table):
    B, C, H, W = inputs.shape
    V, D = table.shape
    P = H * W
    maxv = V // C

    idx = inputs.astype(jnp.int32).reshape(B, C, P)

    tab = table.reshape(C, maxv, D)
    tab = jnp.pad(tab, ((0, 0), (0, KPAD - maxv), (0, 0)))
    tabT = jnp.transpose(tab, (2, 0, 1)).reshape(D, C * KPAD)
    tabT = tabT.astype(jnp.bfloat16)

    T = 16
    grid = (B // T,)
    out = pl.pallas_call(
        functools.partial(_body, t_imgs=T, n_chan=C, kpad=KPAD, pixels=P),
        grid=grid,
        in_specs=[
            pl.BlockSpec((T, C, P), lambda i: (i, 0, 0)),
            pl.BlockSpec((D, C * KPAD), lambda i: (0, 0)),
        ],
        out_specs=pl.BlockSpec((T, D, P), lambda i: (i, 0, 0)),
        out_shape=jax.ShapeDtypeStruct((B, D, P), jnp.float32),
        compiler_params=pltpu.CompilerParams(
            dimension_semantics=("parallel",)),
    )(idx, tabT)
    return out.reshape(B, D, H, W)


# speedup vs baseline: 18.9002x; 1.0140x over previous
import functools

import jax
import jax.numpy as jnp
from jax.experimental import pallas as pl
from jax.experimental.pallas import tpu as pltpu

MAXV = 147
KPAD = 160


def _body(idx_ref, tab_ref, out_ref, *, t_imgs, n_chan, kpad, pixels):
    iota = jax.lax.broadcasted_iota(jnp.int32, (kpad, pixels), 0)
    iota_bf = iota.astype(jnp.bfloat16)
    one = jnp.bfloat16(1.0)
    zero = jnp.bfloat16(0.0)
    for t in range(t_imgs):
        hots = []
        for c in range(n_chan):
            idx_bf = idx_ref[t, c, :].astype(jnp.bfloat16)
            d = iota_bf - idx_bf[None, :]
            hots.append(jnp.maximum(one - jnp.abs(d), zero))
        onehot = jnp.concatenate(hots, axis=0)
        out_ref[t] = jnp.dot(tab_ref[...], onehot,
                             preferred_element_type=jnp.float32)


@jax.jit
def kernel(inputs, table):
    B, C, H, W = inputs.shape
    V, D = table.shape
    P = H * W
    maxv = V // C

    idx = inputs.astype(jnp.int32).reshape(B, C, P)

    tab = table.reshape(C, maxv, D)
    tab = jnp.pad(tab, ((0, 0), (0, KPAD - maxv), (0, 0)))
    tabT = jnp.transpose(tab, (2, 0, 1)).reshape(D, C * KPAD)
    tabT = tabT.astype(jnp.bfloat16)

    T = 32
    grid = (B // T,)
    out = pl.pallas_call(
        functools.partial(_body, t_imgs=T, n_chan=C, kpad=KPAD, pixels=P),
        grid=grid,
        in_specs=[
            pl.BlockSpec((T, C, P), lambda i: (i, 0, 0)),
            pl.BlockSpec((D, C * KPAD), lambda i: (0, 0)),
        ],
        out_specs=pl.BlockSpec((T, D, P), lambda i: (i, 0, 0)),
        out_shape=jax.ShapeDtypeStruct((B, D, P), jnp.float32),
        compiler_params=pltpu.CompilerParams(
            dimension_semantics=("parallel",)),
    )(idx, tabT)
    return out.reshape(B, D, H, W)
